# baseline (speedup 1.0000x reference)
import functools

import jax
import jax.numpy as jnp
from jax import lax
from jax.experimental import pallas as pl
from jax.experimental.pallas import tpu as pltpu
from jax.experimental.pallas import tpu_sc as plsc

B = 16384
NC, NS = 2, 16
NW = NC * NS
NU = 1000000
NV = 50356
UCHUNK = 65536
NUCHUNK = (NU + UCHUNK - 1) // UCHUNK
SMALL_OFF = (0, 9, 13, 47, 56)
SMALL_W = 1888
NV_PAD = 50360

_mesh = plsc.VectorSubcoreMesh(core_axis_name="c", subcore_axis_name="s")


def _gather_all(ids_ref, src_ref, out_ref, row):
    def body(i, _):
        sl = pl.ds(i * 16, 16)
        idx = ids_ref[sl]
        out_ref[sl] = plsc.load_gather(src_ref, [row, idx])
        return 0
    lax.fori_loop(0, B // 16, body, 0, unroll=8)


@functools.partial(
    pl.kernel,
    mesh=_mesh,
    compiler_params=pltpu.CompilerParams(use_tc_tiling_on_sc=False,
                                         needs_layout_passes=False),
    out_type=(
        jax.ShapeDtypeStruct((32, B), jnp.float32),
        jax.ShapeDtypeStruct((32, B), jnp.float32),
        jax.ShapeDtypeStruct((25, B), jnp.float32),
    ),
    scratch_types=(),
)
def _sc_gather(utT, vtT, smT,
               uid, vid, aid, gid, pid, cid, did,
               uT_out, vT_out, sT_out):
    w = lax.axis_index("s") * NC + lax.axis_index("c")

    def video_phase(idsv, vrow, vout):
        pltpu.sync_copy(vid, idsv)
        pltpu.sync_copy(vtT.at[w], vrow)
        def body(i, _):
            sl = pl.ds(i * 16, 16)
            vout[sl] = plsc.load_gather(vrow, [idsv[sl]])
            return 0
        lax.fori_loop(0, B // 16, body, 0, unroll=8)
        pltpu.sync_copy(vout, vT_out.at[w])

    pl.run_scoped(video_phase,
                  pltpu.VMEM((B,), jnp.int32),
                  pltpu.VMEM((NV_PAD,), jnp.float32),
                  pltpu.VMEM((B,), jnp.float32))

    def small_phase(idss, stab, sout):
        pltpu.sync_copy(smT, stab)
        for t, ids_h in enumerate((aid, gid, pid, cid, did)):
            off = SMALL_OFF[t]
            @pl.when(jnp.logical_and(w >= 5 * t, w < 5 * t + 5))
            def _():
                row = jnp.full((16,), w - 5 * t, jnp.int32)
                pltpu.sync_copy(ids_h, idss)
                def body(i, _):
                    sl = pl.ds(i * 16, 16)
                    sout[sl] = plsc.load_gather(stab, [row, idss[sl] + off])
                    return 0
                lax.fori_loop(0, B // 16, body, 0, unroll=8)
                pltpu.sync_copy(sout, sT_out.at[w])

    pl.run_scoped(small_phase,
                  pltpu.VMEM((B,), jnp.int32),
                  pltpu.VMEM((8, SMALL_W), jnp.float32),
                  pltpu.VMEM((B,), jnp.float32))

    def user_phase(idsu, cbuf, uout):
        pltpu.sync_copy(uid, idsu)
        for c in range(NUCHUNK):
            lo = c * UCHUNK
            size = min(UCHUNK, NU - lo)
            pltpu.sync_copy(utT.at[w, pl.ds(lo, size)],
                            cbuf.at[pl.ds(0, size)])
            def body(i, _):
                sl = pl.ds(i * 16, 16)
                ids = idsu[sl]
                m = (ids >> 16) == c
                g = plsc.load_gather(cbuf, [ids & (UCHUNK - 1)], mask=m)
                uout[sl] = jnp.where(m, g, uout[sl])
                return 0
            lax.fori_loop(0, B // 16, body, 0, unroll=8)
        pltpu.sync_copy(uout, uT_out.at[w])

    pl.run_scoped(user_phase,
                  pltpu.VMEM((B,), jnp.int32),
                  pltpu.VMEM((UCHUNK,), jnp.float32),
                  pltpu.VMEM((B,), jnp.float32))


def _tc_body(uT_ref, vT_ref, sT_ref, vsc_ref, vact_ref, vdir_ref, sco_ref,
             dur_ref, Wsec_ref, Wact_ref, Wdir_ref, Wsco_ref, Wdur_ref,
             W1_ref, bsec_ref, bact_ref, bdir_ref, bsco_ref, bdur_ref,
             b1_ref, W2_ref, b2_ref, W3_ref, b3_ref, out_ref):
    f32 = jnp.float32
    dot = functools.partial(jnp.dot, preferred_element_type=f32)
    dotT = lambda a, b: lax.dot_general(a, b, (((0,), (0,)), ((), ())),
                                        preferred_element_type=f32)
    W1 = W1_ref[...]

    h = dotT(uT_ref[...], W1[0:32])
    h += dotT(vT_ref[...], W1[32:64])
    h += dotT(sT_ref[...], W1[81:106])
    h += dot(vsc_ref[...], dot(Wsec_ref[...], W1[64:69]))
    h += dot(vact_ref[...], dot(Wact_ref[...], W1[69:74]))
    h += dot(vdir_ref[...], dot(Wdir_ref[...], W1[74:79]))
    h += dot(sco_ref[...].reshape(-1, 1), dot(Wsco_ref[...], W1[79:80]))
    h += dot(dur_ref[...].reshape(-1, 1), dot(Wdur_ref[...], W1[80:81]))
    bias = b1_ref[...].reshape(1, 64)
    bias += dot(bsec_ref[...].reshape(1, 5), W1[64:69])
    bias += dot(bact_ref[...].reshape(1, 5), W1[69:74])
    bias += dot(bdir_ref[...].reshape(1, 5), W1[74:79])
    bias += bsco_ref[...].reshape(1, 1) * W1[79:80]
    bias += bdur_ref[...].reshape(1, 1) * W1[80:81]
    h = jnp.maximum(h + bias, 0.0)
    h = jnp.maximum(dot(h, W2_ref[...]) + b2_ref[...].reshape(1, 32), 0.0)
    out_ref[...] = dot(h, W3_ref[...]) + b3_ref[...].reshape(1, 10)


def kernel(user_id, video_id, video_second_class, video_actor_list,
           video_director_list, video_score, video_duration, age, gender,
           province, city_level, device_name, user_table, video_table,
           age_table, gender_table, province_table, city_table, device_table,
           W_sec, b_sec, W_act, b_act, W_dir, b_dir, W_score, b_score,
           W_dur, b_dur, W_fc1, b_fc1, W_fc2, b_fc2, W_out, b_out):
    i32 = jnp.int32
    f32 = jnp.float32

    ids = [x.astype(i32) for x in (user_id, video_id, age, gender, province,
                                   city_level, device_name)]
    smT = jnp.pad(
        jnp.concatenate([age_table.T, gender_table.T, province_table.T,
                         city_table.T, device_table.T], axis=1),
        ((0, 3), (0, SMALL_W - 1883)))
    vtT = jnp.pad(video_table.T, ((0, 0), (0, NV_PAD - NV)))
    uT_out, vT_out, sT_out = _sc_gather(user_table.T, vtT, smT, *ids)

    blk = 4096
    grid = (B // blk,)

    ins = (uT_out, vT_out, sT_out, video_second_class, video_actor_list,
           video_director_list, video_score, video_duration,
           W_sec, W_act, W_dir, W_score, W_dur, W_fc1,
           b_sec, b_act, b_dir, b_score, b_dur, b_fc1, W_fc2, b_fc2,
           W_out, b_out)
    in_specs = [
        pl.BlockSpec((32, blk), lambda i: (0, i)),
        pl.BlockSpec((32, blk), lambda i: (0, i)),
        pl.BlockSpec((25, blk), lambda i: (0, i)),
        pl.BlockSpec((blk, 5), lambda i: (i, 0)),
        pl.BlockSpec((blk, 5), lambda i: (i, 0)),
        pl.BlockSpec((blk, 5), lambda i: (i, 0)),
        pl.BlockSpec((blk,), lambda i: (i,)),
        pl.BlockSpec((blk,), lambda i: (i,)),
    ] + [pl.BlockSpec(a.shape, lambda i, _n=a.ndim: (0,) * _n)
         for a in ins[8:]]

    return pl.pallas_call(
        _tc_body,
        grid=grid,
        in_specs=in_specs,
        out_specs=pl.BlockSpec((blk, 10), lambda i: (i, 0)),
        out_shape=jax.ShapeDtypeStruct((B, 10), f32),
    )(*ins)


# speedup vs baseline: 4.7158x; 4.7158x over previous
import functools

import jax
import jax.numpy as jnp
from jax import lax
from jax.experimental import pallas as pl
from jax.experimental.pallas import tpu as pltpu
from jax.experimental.pallas import tpu_sc as plsc

B = 16384
NC, NS = 2, 16
NW = NC * NS
BPW = B // NW
CHUNK = 128
NCHUNK = BPW // CHUNK
NU = 1000000
USTRIDE = 250880
NUROWS = USTRIDE
NV = 50356
SPAD = 16
SMALL_OFF = (0, 9, 13, 47, 56)
SMALL_ROWS = 1888

TRR = 1024
TRGRID = NUROWS // TRR
_ULASTBLK = (NU - 1) // TRR

_mesh = plsc.VectorSubcoreMesh(core_axis_name="c", subcore_axis_name="s")


def _tr_body(in0, in1, in2, in3, out_ref):
    for q, ref in enumerate((in0, in1, in2, in3)):
        out_ref[:, 32 * q:32 * q + 32] = ref[...].T


def _pack_user(utT):
    spec = lambda q: pl.BlockSpec(
        (32, TRR), lambda i, _q=q: (0, jnp.minimum(i + _q * TRGRID, _ULASTBLK)))
    return pl.pallas_call(
        _tr_body,
        grid=(TRGRID,),
        in_specs=[spec(0), spec(1), spec(2), spec(3)],
        out_specs=pl.BlockSpec((TRR, 128), lambda i: (i, 0)),
        out_shape=jax.ShapeDtypeStruct((NUROWS, 128), jnp.float32),
    )(utT, utT, utT, utT)


@functools.partial(
    pl.kernel,
    mesh=_mesh,
    compiler_params=pltpu.CompilerParams(use_tc_tiling_on_sc=False),
    out_type=(
        jax.ShapeDtypeStruct((B, 128), jnp.float32),
        jax.ShapeDtypeStruct((B, 32), jnp.float32),
        jax.ShapeDtypeStruct((5, B, SPAD), jnp.float32),
    ),
    scratch_types=(
        [pltpu.VMEM((NCHUNK, CHUNK), jnp.int32) for _ in range(7)]
        + [
            pltpu.VMEM((BPW, 128), jnp.float32),
            pltpu.VMEM((BPW, 32), jnp.float32),
        ]
        + [pltpu.VMEM((BPW, SPAD), jnp.float32) for _ in range(5)]
        + [pltpu.SemaphoreType.DMA, pltpu.SemaphoreType.DMA,
           pltpu.SemaphoreType.DMA]
    ),
)
def _sc_gather(t128, video_table, sm16,
               uid4, vid, aid, gid, pid, cid, did,
               out_u, out_v, out_s,
               ixu, ixv, ix0, ix1, ix2, ix3, ix4,
               ru, rv, rs0, rs1, rs2, rs3, rs4, sem_i, sem_g, sem_o):
    wid = lax.axis_index("s") * NC + lax.axis_index("c")
    base = wid * BPW
    crow = wid * NCHUNK

    idx_refs = (ixu, ixv, ix0, ix1, ix2, ix3, ix4)
    id_hbm = (uid4, vid, aid, gid, pid, cid, did)
    tables = (t128, video_table, sm16, sm16, sm16, sm16, sm16)
    rows = (ru, rv, rs0, rs1, rs2, rs3, rs4)
    outs = (out_u.at[pl.ds(base, BPW)], out_v.at[pl.ds(base, BPW)]) + tuple(
        out_s.at[t, pl.ds(base, BPW)] for t in range(5))

    stage = [pltpu.async_copy(ids.at[pl.ds(crow, NCHUNK)], ix, sem_i)
             for ix, ids in zip(idx_refs, id_hbm)]
    for c in stage:
        c.wait()

    gath = [pltpu.async_copy(tab.at[ix.at[j]],
                             dst.at[pl.ds(j * CHUNK, CHUNK)], sem_g)
            for ix, tab, dst in zip(idx_refs, tables, rows)
            for j in range(NCHUNK)]
    for c in gath:
        c.wait()
    wb = [pltpu.async_copy(src, dst, sem_o) for src, dst in zip(rows, outs)]
    for c in wb:
        c.wait()


def _tc_body(u_ref, uid_ref, v_ref, s_ref, vsc_ref, vact_ref, vdir_ref,
             sco_ref, dur_ref, Wsec_ref, Wact_ref, Wdir_ref, Wsco_ref,
             Wdur_ref, W1u4_ref, W1v_ref, W1sec_ref, W1act_ref, W1dir_ref,
             w1sco_ref, w1dur_ref, W1s_ref, bsec_ref, bact_ref, bdir_ref,
             bsco_ref, bdur_ref, b1_ref, W2_ref, b2_ref, W3_ref, b3_ref,
             out_ref):
    f32 = jnp.float32
    dot = functools.partial(jnp.dot, preferred_element_type=f32)

    u128 = u_ref[...]
    q = uid_ref[...]
    lanes = lax.broadcasted_iota(jnp.int32, u128.shape, 1) >> 5
    u_masked = jnp.where(lanes == q, u128, 0.0)
    h = dot(u_masked, W1u4_ref[...])
    h += dot(v_ref[...], W1v_ref[...])
    for t in range(5):
        h += dot(s_ref[t], W1s_ref[t])
    h += dot(vsc_ref[...], dot(Wsec_ref[...], W1sec_ref[...]))
    h += dot(vact_ref[...], dot(Wact_ref[...], W1act_ref[...]))
    h += dot(vdir_ref[...], dot(Wdir_ref[...], W1dir_ref[...]))
    h += dot(sco_ref[...], dot(Wsco_ref[...], w1sco_ref[...]))
    h += dot(dur_ref[...], dot(Wdur_ref[...], w1dur_ref[...]))
    bias = b1_ref[...]
    bias += dot(bsec_ref[...], W1sec_ref[...])
    bias += dot(bact_ref[...], W1act_ref[...])
    bias += dot(bdir_ref[...], W1dir_ref[...])
    bias += dot(bsco_ref[...], w1sco_ref[...])
    bias += dot(bdur_ref[...], w1dur_ref[...])
    h = jnp.maximum(h + bias, 0.0)
    h = jnp.maximum(dot(h, W2_ref[...]) + b2_ref[...], 0.0)
    out_ref[...] = dot(h, W3_ref[...]) + b3_ref[...]


def kernel(user_id, video_id, video_second_class, video_actor_list,
           video_director_list, video_score, video_duration, age, gender,
           province, city_level, device_name, user_table, video_table,
           age_table, gender_table, province_table, city_table, device_table,
           W_sec, b_sec, W_act, b_act, W_dir, b_dir, W_score, b_score,
           W_dur, b_dur, W_fc1, b_fc1, W_fc2, b_fc2, W_out, b_out):
    i32 = jnp.int32
    f32 = jnp.float32

    t128 = _pack_user(user_table.T)
    sm16 = jnp.pad(
        jnp.concatenate([age_table, gender_table, province_table, city_table,
                         device_table], axis=0),
        ((0, SMALL_ROWS - 1883), (0, SPAD - 5)))

    uid = user_id.astype(i32)
    ids2d = [x.reshape(B // CHUNK, CHUNK) for x in (
        uid % USTRIDE,
        video_id.astype(i32),
        age.astype(i32) + SMALL_OFF[0],
        gender.astype(i32) + SMALL_OFF[1],
        province.astype(i32) + SMALL_OFF[2],
        city_level.astype(i32) + SMALL_OFF[3],
        device_name.astype(i32) + SMALL_OFF[4],
    )]

    out_u, out_v, out_s = _sc_gather(t128, video_table, sm16, *ids2d)

    W1u4 = jnp.concatenate([W_fc1[0:32]] * 4, axis=0)
    W1v = W_fc1[32:64]
    W1sec = W_fc1[64:69]
    W1act = W_fc1[69:74]
    W1dir = W_fc1[74:79]
    w1sco = W_fc1[79:80]
    w1dur = W_fc1[80:81]
    W1s = jnp.stack([jnp.pad(W_fc1[81 + 5 * t:86 + 5 * t],
                             ((0, SPAD - 5), (0, 0))) for t in range(5)])

    blk = 2048
    grid = (B // blk,)
    ins = (out_u, (uid // USTRIDE).reshape(B, 1), out_v, out_s,
           video_second_class, video_actor_list, video_director_list,
           video_score.reshape(B, 1), video_duration.reshape(B, 1),
           W_sec, W_act, W_dir, W_score, W_dur,
           W1u4, W1v, W1sec, W1act, W1dir, w1sco, w1dur, W1s,
           b_sec.reshape(1, 5), b_act.reshape(1, 5), b_dir.reshape(1, 5),
           b_score.reshape(1, 1), b_dur.reshape(1, 1), b_fc1.reshape(1, 64),
           W_fc2, b_fc2.reshape(1, 32), W_out, b_out.reshape(1, 10))
    in_specs = [
        pl.BlockSpec((blk, 128), lambda i: (i, 0)),
        pl.BlockSpec((blk, 1), lambda i: (i, 0)),
        pl.BlockSpec((blk, 32), lambda i: (i, 0)),
        pl.BlockSpec((5, blk, SPAD), lambda i: (0, i, 0)),
        pl.BlockSpec((blk, 5), lambda i: (i, 0)),
        pl.BlockSpec((blk, 5), lambda i: (i, 0)),
        pl.BlockSpec((blk, 5), lambda i: (i, 0)),
        pl.BlockSpec((blk, 1), lambda i: (i, 0)),
        pl.BlockSpec((blk, 1), lambda i: (i, 0)),
    ] + [pl.BlockSpec(a.shape, lambda i, _n=a.ndim: (0,) * _n)
         for a in ins[9:]]

    return pl.pallas_call(
        _tc_body,
        grid=grid,
        in_specs=in_specs,
        out_specs=pl.BlockSpec((blk, 10), lambda i: (i, 0)),
        out_shape=jax.ShapeDtypeStruct((B, 10), f32),
    )(*ins)


# speedup vs baseline: 4.8954x; 1.0381x over previous
import functools

import jax
import jax.numpy as jnp
from jax import lax
from jax.experimental import pallas as pl
from jax.experimental.pallas import tpu as pltpu
from jax.experimental.pallas import tpu_sc as plsc

B = 16384
NC, NS = 2, 16
NW = NC * NS
BPW = B // NW
CHUNK = 128
NCHUNK = BPW // CHUNK
NU = 1000000
USTRIDE = 250880
NUROWS = USTRIDE
NV = 50356
SPAD = 16
SMALL_OFF = (0, 9, 13, 47, 56)
SMALL_ROWS = 1888

TRR = 1024
TRGRID = NUROWS // TRR
_ULASTBLK = (NU - 1) // TRR

_mesh = plsc.VectorSubcoreMesh(core_axis_name="c", subcore_axis_name="s")


def _tr_body(in0, in1, in2, in3, out_ref):
    f32 = jnp.float32
    eye = jnp.eye(128, dtype=f32)
    acc = None
    for q, ref in enumerate((in0, in1, in2, in3)):
        c = lax.dot_general(ref[...], eye[32 * q:32 * q + 32, :],
                            (((0,), (0,)), ((), ())),
                            preferred_element_type=f32)
        acc = c if acc is None else acc + c
    out_ref[...] = acc


def _pack_user(utT):
    spec = lambda q: pl.BlockSpec(
        (32, TRR), lambda i, _q=q: (0, jnp.minimum(i + _q * TRGRID, _ULASTBLK)))
    return pl.pallas_call(
        _tr_body,
        grid=(TRGRID,),
        in_specs=[spec(0), spec(1), spec(2), spec(3)],
        out_specs=pl.BlockSpec((TRR, 128), lambda i: (i, 0)),
        out_shape=jax.ShapeDtypeStruct((NUROWS, 128), jnp.float32),
    )(utT, utT, utT, utT)


@functools.partial(
    pl.kernel,
    mesh=_mesh,
    compiler_params=pltpu.CompilerParams(use_tc_tiling_on_sc=False),
    out_type=(
        jax.ShapeDtypeStruct((B, 128), jnp.float32),
        jax.ShapeDtypeStruct((B, 32), jnp.float32),
        jax.ShapeDtypeStruct((5, B, SPAD), jnp.float32),
    ),
    scratch_types=(
        [pltpu.VMEM((NCHUNK, CHUNK), jnp.int32) for _ in range(7)]
        + [
            pltpu.VMEM((BPW, 128), jnp.float32),
            pltpu.VMEM((BPW, 32), jnp.float32),
        ]
        + [pltpu.VMEM((BPW, SPAD), jnp.float32) for _ in range(5)]
        + [pltpu.SemaphoreType.DMA, pltpu.SemaphoreType.DMA,
           pltpu.SemaphoreType.DMA]
    ),
)
def _sc_gather(t128, video_table, sm16,
               uid4, vid, aid, gid, pid, cid, did,
               out_u, out_v, out_s,
               ixu, ixv, ix0, ix1, ix2, ix3, ix4,
               ru, rv, rs0, rs1, rs2, rs3, rs4, sem_i, sem_g, sem_o):
    wid = lax.axis_index("s") * NC + lax.axis_index("c")
    base = wid * BPW
    crow = wid * NCHUNK

    idx_refs = (ixu, ixv, ix0, ix1, ix2, ix3, ix4)
    id_hbm = (uid4, vid, aid, gid, pid, cid, did)
    tables = (t128, video_table, sm16, sm16, sm16, sm16, sm16)
    rows = (ru, rv, rs0, rs1, rs2, rs3, rs4)
    outs = (out_u.at[pl.ds(base, BPW)], out_v.at[pl.ds(base, BPW)]) + tuple(
        out_s.at[t, pl.ds(base, BPW)] for t in range(5))

    stage = [pltpu.async_copy(ids.at[pl.ds(crow, NCHUNK)], ix, sem_i)
             for ix, ids in zip(idx_refs, id_hbm)]
    for c in stage:
        c.wait()

    gath = [pltpu.async_copy(tab.at[ix.at[j]],
                             dst.at[pl.ds(j * CHUNK, CHUNK)], sem_g)
            for ix, tab, dst in zip(idx_refs, tables, rows)
            for j in range(NCHUNK)]
    for c in gath:
        c.wait()
    wb = [pltpu.async_copy(src, dst, sem_o) for src, dst in zip(rows, outs)]
    for c in wb:
        c.wait()


def _tc_body(u_ref, uid_ref, v_ref, s_ref, vsc_ref, vact_ref, vdir_ref,
             sco_ref, dur_ref, Wsec_ref, Wact_ref, Wdir_ref, Wsco_ref,
             Wdur_ref, W1u4_ref, W1v_ref, W1sec_ref, W1act_ref, W1dir_ref,
             w1sco_ref, w1dur_ref, W1s_ref, bsec_ref, bact_ref, bdir_ref,
             bsco_ref, bdur_ref, b1_ref, W2_ref, b2_ref, W3_ref, b3_ref,
             out_ref):
    f32 = jnp.float32
    dot = functools.partial(jnp.dot, preferred_element_type=f32)

    u128 = u_ref[...]
    q = uid_ref[...]
    lanes = lax.broadcasted_iota(jnp.int32, u128.shape, 1) >> 5
    u_masked = jnp.where(lanes == q, u128, 0.0)
    h = dot(u_masked, W1u4_ref[...])
    h += dot(v_ref[...], W1v_ref[...])
    for t in range(5):
        h += dot(s_ref[t], W1s_ref[t])
    h += dot(vsc_ref[...], dot(Wsec_ref[...], W1sec_ref[...]))
    h += dot(vact_ref[...], dot(Wact_ref[...], W1act_ref[...]))
    h += dot(vdir_ref[...], dot(Wdir_ref[...], W1dir_ref[...]))
    h += dot(sco_ref[...], dot(Wsco_ref[...], w1sco_ref[...]))
    h += dot(dur_ref[...], dot(Wdur_ref[...], w1dur_ref[...]))
    bias = b1_ref[...]
    bias += dot(bsec_ref[...], W1sec_ref[...])
    bias += dot(bact_ref[...], W1act_ref[...])
    bias += dot(bdir_ref[...], W1dir_ref[...])
    bias += dot(bsco_ref[...], w1sco_ref[...])
    bias += dot(bdur_ref[...], w1dur_ref[...])
    h = jnp.maximum(h + bias, 0.0)
    h = jnp.maximum(dot(h, W2_ref[...]) + b2_ref[...], 0.0)
    out_ref[...] = dot(h, W3_ref[...]) + b3_ref[...]


def kernel(user_id, video_id, video_second_class, video_actor_list,
           video_director_list, video_score, video_duration, age, gender,
           province, city_level, device_name, user_table, video_table,
           age_table, gender_table, province_table, city_table, device_table,
           W_sec, b_sec, W_act, b_act, W_dir, b_dir, W_score, b_score,
           W_dur, b_dur, W_fc1, b_fc1, W_fc2, b_fc2, W_out, b_out):
    i32 = jnp.int32
    f32 = jnp.float32

    t128 = _pack_user(user_table.T)
    sm16 = jnp.pad(
        jnp.concatenate([age_table, gender_table, province_table, city_table,
                         device_table], axis=0),
        ((0, SMALL_ROWS - 1883), (0, SPAD - 5)))

    uid = user_id.astype(i32)
    ids2d = [x.reshape(B // CHUNK, CHUNK) for x in (
        uid % USTRIDE,
        video_id.astype(i32),
        age.astype(i32) + SMALL_OFF[0],
        gender.astype(i32) + SMALL_OFF[1],
        province.astype(i32) + SMALL_OFF[2],
        city_level.astype(i32) + SMALL_OFF[3],
        device_name.astype(i32) + SMALL_OFF[4],
    )]

    out_u, out_v, out_s = _sc_gather(t128, video_table, sm16, *ids2d)

    W1u4 = jnp.concatenate([W_fc1[0:32]] * 4, axis=0)
    W1v = W_fc1[32:64]
    W1sec = W_fc1[64:69]
    W1act = W_fc1[69:74]
    W1dir = W_fc1[74:79]
    w1sco = W_fc1[79:80]
    w1dur = W_fc1[80:81]
    W1s = jnp.stack([jnp.pad(W_fc1[81 + 5 * t:86 + 5 * t],
                             ((0, SPAD - 5), (0, 0))) for t in range(5)])

    blk = 2048
    grid = (B // blk,)
    ins = (out_u, (uid // USTRIDE).reshape(B, 1), out_v, out_s,
           video_second_class, video_actor_list, video_director_list,
           video_score.reshape(B, 1), video_duration.reshape(B, 1),
           W_sec, W_act, W_dir, W_score, W_dur,
           W1u4, W1v, W1sec, W1act, W1dir, w1sco, w1dur, W1s,
           b_sec.reshape(1, 5), b_act.reshape(1, 5), b_dir.reshape(1, 5),
           b_score.reshape(1, 1), b_dur.reshape(1, 1), b_fc1.reshape(1, 64),
           W_fc2, b_fc2.reshape(1, 32), W_out, b_out.reshape(1, 10))
    in_specs = [
        pl.BlockSpec((blk, 128), lambda i: (i, 0)),
        pl.BlockSpec((blk, 1), lambda i: (i, 0)),
        pl.BlockSpec((blk, 32), lambda i: (i, 0)),
        pl.BlockSpec((5, blk, SPAD), lambda i: (0, i, 0)),
        pl.BlockSpec((blk, 5), lambda i: (i, 0)),
        pl.BlockSpec((blk, 5), lambda i: (i, 0)),
        pl.BlockSpec((blk, 5), lambda i: (i, 0)),
        pl.BlockSpec((blk, 1), lambda i: (i, 0)),
        pl.BlockSpec((blk, 1), lambda i: (i, 0)),
    ] + [pl.BlockSpec(a.shape, lambda i, _n=a.ndim: (0,) * _n)
         for a in ins[9:]]

    return pl.pallas_call(
        _tc_body,
        grid=grid,
        in_specs=in_specs,
        out_specs=pl.BlockSpec((blk, 10), lambda i: (i, 0)),
        out_shape=jax.ShapeDtypeStruct((B, 10), f32),
    )(*ins)


# speedup vs baseline: 5.9980x; 1.2252x over previous
import functools

import jax
import jax.numpy as jnp
from jax import lax
from jax.experimental import pallas as pl
from jax.experimental.pallas import tpu as pltpu
from jax.experimental.pallas import tpu_sc as plsc

B = 16384
NC, NS = 2, 16
NW = NC * NS
BPW = B // NW
CHUNK = 128
NCHUNK = BPW // CHUNK
NU = 1000000
USTRIDE = 253952
NUROWS = USTRIDE
NV = 50356
SPAD = 16
SMALL_OFF = (0, 9, 13, 47, 56)
SMALL_ROWS = 1888

TRR = 4096
TRGRID = NUROWS // TRR
_ULASTBLK = (NU - 1) // TRR

_mesh = plsc.VectorSubcoreMesh(core_axis_name="c", subcore_axis_name="s")


def _tr_body(in0, in1, in2, in3, out_ref):
    f32 = jnp.float32
    eye = jnp.eye(128, dtype=f32)
    acc = None
    for q, ref in enumerate((in0, in1, in2, in3)):
        c = lax.dot_general(ref[...], eye[32 * q:32 * q + 32, :],
                            (((0,), (0,)), ((), ())),
                            preferred_element_type=f32)
        acc = c if acc is None else acc + c
    out_ref[...] = acc


def _pack_user(utT):
    spec = lambda q: pl.BlockSpec(
        (32, TRR), lambda i, _q=q: (0, jnp.minimum(i + _q * TRGRID, _ULASTBLK)))
    return pl.pallas_call(
        _tr_body,
        grid=(TRGRID,),
        in_specs=[spec(0), spec(1), spec(2), spec(3)],
        out_specs=pl.BlockSpec((TRR, 128), lambda i: (i, 0)),
        out_shape=jax.ShapeDtypeStruct((NUROWS, 128), jnp.float32),
    )(utT, utT, utT, utT)


@functools.partial(
    pl.kernel,
    mesh=_mesh,
    compiler_params=pltpu.CompilerParams(use_tc_tiling_on_sc=False),
    out_type=(
        jax.ShapeDtypeStruct((B, 128), jnp.float32),
        jax.ShapeDtypeStruct((B, 32), jnp.float32),
        jax.ShapeDtypeStruct((5, B, SPAD), jnp.float32),
    ),
    scratch_types=(
        [pltpu.VMEM((NCHUNK, CHUNK), jnp.int32) for _ in range(7)]
        + [
            pltpu.VMEM((BPW, 128), jnp.float32),
            pltpu.VMEM((BPW, 32), jnp.float32),
        ]
        + [pltpu.VMEM((BPW, SPAD), jnp.float32) for _ in range(5)]
        + [pltpu.SemaphoreType.DMA, pltpu.SemaphoreType.DMA,
           pltpu.SemaphoreType.DMA]
    ),
)
def _sc_gather(t128, video_table, sm16,
               uid4, vid, aid, gid, pid, cid, did,
               out_u, out_v, out_s,
               ixu, ixv, ix0, ix1, ix2, ix3, ix4,
               ru, rv, rs0, rs1, rs2, rs3, rs4, sem_i, sem_g, sem_o):
    wid = lax.axis_index("s") * NC + lax.axis_index("c")
    base = wid * BPW
    crow = wid * NCHUNK

    idx_refs = (ixu, ixv, ix0, ix1, ix2, ix3, ix4)
    id_hbm = (uid4, vid, aid, gid, pid, cid, did)
    tables = (t128, video_table, sm16, sm16, sm16, sm16, sm16)
    rows = (ru, rv, rs0, rs1, rs2, rs3, rs4)
    outs = (out_u.at[pl.ds(base, BPW)], out_v.at[pl.ds(base, BPW)]) + tuple(
        out_s.at[t, pl.ds(base, BPW)] for t in range(5))

    stage = [pltpu.async_copy(ids.at[pl.ds(crow, NCHUNK)], ix, sem_i)
             for ix, ids in zip(idx_refs, id_hbm)]
    for c in stage:
        c.wait()

    gath = [pltpu.async_copy(tab.at[ix.at[j]],
                             dst.at[pl.ds(j * CHUNK, CHUNK)], sem_g)
            for ix, tab, dst in zip(idx_refs, tables, rows)
            for j in range(NCHUNK)]
    for c in gath:
        c.wait()
    wb = [pltpu.async_copy(src, dst, sem_o) for src, dst in zip(rows, outs)]
    for c in wb:
        c.wait()


def _tc_body(u_ref, uid_ref, v_ref, s_ref, vsc_ref, vact_ref, vdir_ref,
             sco_ref, dur_ref, Wsec_ref, Wact_ref, Wdir_ref, Wsco_ref,
             Wdur_ref, W1u4_ref, W1v_ref, W1sec_ref, W1act_ref, W1dir_ref,
             w1sco_ref, w1dur_ref, W1s_ref, bsec_ref, bact_ref, bdir_ref,
             bsco_ref, bdur_ref, b1_ref, W2_ref, b2_ref, W3_ref, b3_ref,
             out_ref):
    f32 = jnp.float32
    dot = functools.partial(jnp.dot, preferred_element_type=f32)

    u128 = u_ref[...]
    q = uid_ref[...]
    lanes = lax.broadcasted_iota(jnp.int32, u128.shape, 1) >> 5
    u_masked = jnp.where(lanes == q, u128, 0.0)
    h = dot(u_masked, W1u4_ref[...])
    h += dot(v_ref[...], W1v_ref[...])
    for t in range(5):
        h += dot(s_ref[t], W1s_ref[t])
    h += dot(vsc_ref[...], dot(Wsec_ref[...], W1sec_ref[...]))
    h += dot(vact_ref[...], dot(Wact_ref[...], W1act_ref[...]))
    h += dot(vdir_ref[...], dot(Wdir_ref[...], W1dir_ref[...]))
    h += dot(sco_ref[...], dot(Wsco_ref[...], w1sco_ref[...]))
    h += dot(dur_ref[...], dot(Wdur_ref[...], w1dur_ref[...]))
    bias = b1_ref[...]
    bias += dot(bsec_ref[...], W1sec_ref[...])
    bias += dot(bact_ref[...], W1act_ref[...])
    bias += dot(bdir_ref[...], W1dir_ref[...])
    bias += dot(bsco_ref[...], w1sco_ref[...])
    bias += dot(bdur_ref[...], w1dur_ref[...])
    h = jnp.maximum(h + bias, 0.0)
    h = jnp.maximum(dot(h, W2_ref[...]) + b2_ref[...], 0.0)
    out_ref[...] = dot(h, W3_ref[...]) + b3_ref[...]


def kernel(user_id, video_id, video_second_class, video_actor_list,
           video_director_list, video_score, video_duration, age, gender,
           province, city_level, device_name, user_table, video_table,
           age_table, gender_table, province_table, city_table, device_table,
           W_sec, b_sec, W_act, b_act, W_dir, b_dir, W_score, b_score,
           W_dur, b_dur, W_fc1, b_fc1, W_fc2, b_fc2, W_out, b_out):
    i32 = jnp.int32
    f32 = jnp.float32

    t128 = _pack_user(user_table.T)
    sm16 = jnp.pad(
        jnp.concatenate([age_table, gender_table, province_table, city_table,
                         device_table], axis=0),
        ((0, SMALL_ROWS - 1883), (0, SPAD - 5)))

    uid = user_id.astype(i32)
    ids2d = [x.reshape(B // CHUNK, CHUNK) for x in (
        uid % USTRIDE,
        video_id.astype(i32),
        age.astype(i32) + SMALL_OFF[0],
        gender.astype(i32) + SMALL_OFF[1],
        province.astype(i32) + SMALL_OFF[2],
        city_level.astype(i32) + SMALL_OFF[3],
        device_name.astype(i32) + SMALL_OFF[4],
    )]

    out_u, out_v, out_s = _sc_gather(t128, video_table, sm16, *ids2d)

    W1u4 = jnp.concatenate([W_fc1[0:32]] * 4, axis=0)
    W1v = W_fc1[32:64]
    W1sec = W_fc1[64:69]
    W1act = W_fc1[69:74]
    W1dir = W_fc1[74:79]
    w1sco = W_fc1[79:80]
    w1dur = W_fc1[80:81]
    W1s = jnp.stack([jnp.pad(W_fc1[81 + 5 * t:86 + 5 * t],
                             ((0, SPAD - 5), (0, 0))) for t in range(5)])

    blk = 2048
    grid = (B // blk,)
    ins = (out_u, (uid // USTRIDE).reshape(B, 1), out_v, out_s,
           video_second_class, video_actor_list, video_director_list,
           video_score.reshape(B, 1), video_duration.reshape(B, 1),
           W_sec, W_act, W_dir, W_score, W_dur,
           W1u4, W1v, W1sec, W1act, W1dir, w1sco, w1dur, W1s,
           b_sec.reshape(1, 5), b_act.reshape(1, 5), b_dir.reshape(1, 5),
           b_score.reshape(1, 1), b_dur.reshape(1, 1), b_fc1.reshape(1, 64),
           W_fc2, b_fc2.reshape(1, 32), W_out, b_out.reshape(1, 10))
    in_specs = [
        pl.BlockSpec((blk, 128), lambda i: (i, 0)),
        pl.BlockSpec((blk, 1), lambda i: (i, 0)),
        pl.BlockSpec((blk, 32), lambda i: (i, 0)),
        pl.BlockSpec((5, blk, SPAD), lambda i: (0, i, 0)),
        pl.BlockSpec((blk, 5), lambda i: (i, 0)),
        pl.BlockSpec((blk, 5), lambda i: (i, 0)),
        pl.BlockSpec((blk, 5), lambda i: (i, 0)),
        pl.BlockSpec((blk, 1), lambda i: (i, 0)),
        pl.BlockSpec((blk, 1), lambda i: (i, 0)),
    ] + [pl.BlockSpec(a.shape, lambda i, _n=a.ndim: (0,) * _n)
         for a in ins[9:]]

    return pl.pallas_call(
        _tc_body,
        grid=grid,
        in_specs=in_specs,
        out_specs=pl.BlockSpec((blk, 10), lambda i: (i, 0)),
        out_shape=jax.ShapeDtypeStruct((B, 10), f32),
    )(*ins)


# speedup vs baseline: 6.1286x; 1.0218x over previous
import functools

import jax
import jax.numpy as jnp
from jax import lax
from jax.experimental import pallas as pl
from jax.experimental.pallas import tpu as pltpu
from jax.experimental.pallas import tpu_sc as plsc

B = 16384
NC, NS = 2, 16
NW = NC * NS
BPW = B // NW
CHUNK = 128
NCHUNK = BPW // CHUNK
NU = 1000000
USTRIDE = 253952
NUROWS = USTRIDE
NV = 50356
SPAD = 16
SMALL_OFF = (0, 9, 13, 47, 56)
SMALL_ROWS = 1888

TRR = 8192
TRGRID = NUROWS // TRR
_ULASTBLK = (NU - 1) // TRR

_mesh = plsc.VectorSubcoreMesh(core_axis_name="c", subcore_axis_name="s")


def _tr_body(in0, in1, in2, in3, out_ref):
    f32 = jnp.float32
    eye = jnp.eye(128, dtype=f32)
    acc = None
    for q, ref in enumerate((in0, in1, in2, in3)):
        c = lax.dot_general(ref[...], eye[32 * q:32 * q + 32, :],
                            (((0,), (0,)), ((), ())),
                            preferred_element_type=f32)
        acc = c if acc is None else acc + c
    out_ref[...] = acc


def _pack_user(utT):
    spec = lambda q: pl.BlockSpec(
        (32, TRR), lambda i, _q=q: (0, jnp.minimum(i + _q * TRGRID, _ULASTBLK)))
    return pl.pallas_call(
        _tr_body,
        grid=(TRGRID,),
        in_specs=[spec(0), spec(1), spec(2), spec(3)],
        out_specs=pl.BlockSpec((TRR, 128), lambda i: (i, 0)),
        out_shape=jax.ShapeDtypeStruct((NUROWS, 128), jnp.float32),
    )(utT, utT, utT, utT)


@functools.partial(
    pl.kernel,
    mesh=_mesh,
    compiler_params=pltpu.CompilerParams(use_tc_tiling_on_sc=False),
    out_type=(
        jax.ShapeDtypeStruct((B, 128), jnp.float32),
        jax.ShapeDtypeStruct((B, 32), jnp.float32),
        jax.ShapeDtypeStruct((5, B, SPAD), jnp.float32),
    ),
    scratch_types=(
        [pltpu.VMEM((NCHUNK, CHUNK), jnp.int32) for _ in range(7)]
        + [
            pltpu.VMEM((BPW, 128), jnp.float32),
            pltpu.VMEM((BPW, 32), jnp.float32),
        ]
        + [pltpu.VMEM((BPW, SPAD), jnp.float32) for _ in range(5)]
        + [pltpu.SemaphoreType.DMA, pltpu.SemaphoreType.DMA,
           pltpu.SemaphoreType.DMA]
    ),
)
def _sc_gather(t128, video_table, sm16,
               uid4, vid, aid, gid, pid, cid, did,
               out_u, out_v, out_s,
               ixu, ixv, ix0, ix1, ix2, ix3, ix4,
               ru, rv, rs0, rs1, rs2, rs3, rs4, sem_i, sem_g, sem_o):
    wid = lax.axis_index("s") * NC + lax.axis_index("c")
    base = wid * BPW
    crow = wid * NCHUNK

    idx_refs = (ixu, ixv, ix0, ix1, ix2, ix3, ix4)
    id_hbm = (uid4, vid, aid, gid, pid, cid, did)
    tables = (t128, video_table, sm16, sm16, sm16, sm16, sm16)
    rows = (ru, rv, rs0, rs1, rs2, rs3, rs4)
    outs = (out_u.at[pl.ds(base, BPW)], out_v.at[pl.ds(base, BPW)]) + tuple(
        out_s.at[t, pl.ds(base, BPW)] for t in range(5))

    stage = [pltpu.async_copy(ids.at[pl.ds(crow, NCHUNK)], ix, sem_i)
             for ix, ids in zip(idx_refs, id_hbm)]
    for c in stage:
        c.wait()

    gath = [pltpu.async_copy(tab.at[ix.at[j]],
                             dst.at[pl.ds(j * CHUNK, CHUNK)], sem_g)
            for ix, tab, dst in zip(idx_refs, tables, rows)
            for j in range(NCHUNK)]
    for c in gath:
        c.wait()
    wb = [pltpu.async_copy(src, dst, sem_o) for src, dst in zip(rows, outs)]
    for c in wb:
        c.wait()


def _tc_body(u_ref, uid_ref, v_ref, s_ref, vsc_ref, vact_ref, vdir_ref,
             sco_ref, dur_ref, Wsec_ref, Wact_ref, Wdir_ref, Wsco_ref,
             Wdur_ref, W1u4_ref, W1v_ref, W1sec_ref, W1act_ref, W1dir_ref,
             w1sco_ref, w1dur_ref, W1s_ref, bsec_ref, bact_ref, bdir_ref,
             bsco_ref, bdur_ref, b1_ref, W2_ref, b2_ref, W3_ref, b3_ref,
             out_ref):
    f32 = jnp.float32
    dot = functools.partial(jnp.dot, preferred_element_type=f32)

    u128 = u_ref[...]
    q = uid_ref[...]
    lanes = lax.broadcasted_iota(jnp.int32, u128.shape, 1) >> 5
    u_masked = jnp.where(lanes == q, u128, 0.0)
    h = dot(u_masked, W1u4_ref[...])
    h += dot(v_ref[...], W1v_ref[...])
    for t in range(5):
        h += dot(s_ref[t], W1s_ref[t])
    h += dot(vsc_ref[...], dot(Wsec_ref[...], W1sec_ref[...]))
    h += dot(vact_ref[...], dot(Wact_ref[...], W1act_ref[...]))
    h += dot(vdir_ref[...], dot(Wdir_ref[...], W1dir_ref[...]))
    h += dot(sco_ref[...], dot(Wsco_ref[...], w1sco_ref[...]))
    h += dot(dur_ref[...], dot(Wdur_ref[...], w1dur_ref[...]))
    bias = b1_ref[...]
    bias += dot(bsec_ref[...], W1sec_ref[...])
    bias += dot(bact_ref[...], W1act_ref[...])
    bias += dot(bdir_ref[...], W1dir_ref[...])
    bias += dot(bsco_ref[...], w1sco_ref[...])
    bias += dot(bdur_ref[...], w1dur_ref[...])
    h = jnp.maximum(h + bias, 0.0)
    h = jnp.maximum(dot(h, W2_ref[...]) + b2_ref[...], 0.0)
    out_ref[...] = dot(h, W3_ref[...]) + b3_ref[...]


def kernel(user_id, video_id, video_second_class, video_actor_list,
           video_director_list, video_score, video_duration, age, gender,
           province, city_level, device_name, user_table, video_table,
           age_table, gender_table, province_table, city_table, device_table,
           W_sec, b_sec, W_act, b_act, W_dir, b_dir, W_score, b_score,
           W_dur, b_dur, W_fc1, b_fc1, W_fc2, b_fc2, W_out, b_out):
    i32 = jnp.int32
    f32 = jnp.float32

    t128 = _pack_user(user_table.T)
    sm16 = jnp.pad(
        jnp.concatenate([age_table, gender_table, province_table, city_table,
                         device_table], axis=0),
        ((0, SMALL_ROWS - 1883), (0, SPAD - 5)))

    uid = user_id.astype(i32)
    ids2d = [x.reshape(B // CHUNK, CHUNK) for x in (
        uid % USTRIDE,
        video_id.astype(i32),
        age.astype(i32) + SMALL_OFF[0],
        gender.astype(i32) + SMALL_OFF[1],
        province.astype(i32) + SMALL_OFF[2],
        city_level.astype(i32) + SMALL_OFF[3],
        device_name.astype(i32) + SMALL_OFF[4],
    )]

    out_u, out_v, out_s = _sc_gather(t128, video_table, sm16, *ids2d)

    W1u4 = jnp.concatenate([W_fc1[0:32]] * 4, axis=0)
    W1v = W_fc1[32:64]
    W1sec = W_fc1[64:69]
    W1act = W_fc1[69:74]
    W1dir = W_fc1[74:79]
    w1sco = W_fc1[79:80]
    w1dur = W_fc1[80:81]
    W1s = jnp.stack([jnp.pad(W_fc1[81 + 5 * t:86 + 5 * t],
                             ((0, SPAD - 5), (0, 0))) for t in range(5)])

    blk = 2048
    grid = (B // blk,)
    ins = (out_u, (uid // USTRIDE).reshape(B, 1), out_v, out_s,
           video_second_class, video_actor_list, video_director_list,
           video_score.reshape(B, 1), video_duration.reshape(B, 1),
           W_sec, W_act, W_dir, W_score, W_dur,
           W1u4, W1v, W1sec, W1act, W1dir, w1sco, w1dur, W1s,
           b_sec.reshape(1, 5), b_act.reshape(1, 5), b_dir.reshape(1, 5),
           b_score.reshape(1, 1), b_dur.reshape(1, 1), b_fc1.reshape(1, 64),
           W_fc2, b_fc2.reshape(1, 32), W_out, b_out.reshape(1, 10))
    in_specs = [
        pl.BlockSpec((blk, 128), lambda i: (i, 0)),
        pl.BlockSpec((blk, 1), lambda i: (i, 0)),
        pl.BlockSpec((blk, 32), lambda i: (i, 0)),
        pl.BlockSpec((5, blk, SPAD), lambda i: (0, i, 0)),
        pl.BlockSpec((blk, 5), lambda i: (i, 0)),
        pl.BlockSpec((blk, 5), lambda i: (i, 0)),
        pl.BlockSpec((blk, 5), lambda i: (i, 0)),
        pl.BlockSpec((blk, 1), lambda i: (i, 0)),
        pl.BlockSpec((blk, 1), lambda i: (i, 0)),
    ] + [pl.BlockSpec(a.shape, lambda i, _n=a.ndim: (0,) * _n)
         for a in ins[9:]]

    return pl.pallas_call(
        _tc_body,
        grid=grid,
        in_specs=in_specs,
        out_specs=pl.BlockSpec((blk, 10), lambda i: (i, 0)),
        out_shape=jax.ShapeDtypeStruct((B, 10), f32),
    )(*ins)
